# baseline (speedup 1.0000x reference)
import jax
import jax.numpy as jnp
from jax import lax
from jax.experimental import pallas as pl
from jax.experimental.pallas import tpu as pltpu
from jax.experimental.pallas import tpu_sc as plsc

_LANES = 16
_NBUF = 3
_S_BLK = 8
_N_SUBCORES = 32


def kernel(x, emb_table):
    B, S, D = x.shape
    pos = emb_table[:S]
    n_step = S // _S_BLK // _N_SUBCORES

    vector_mesh = plsc.VectorSubcoreMesh(
        core_axis_name="core", subcore_axis_name="subcore"
    )

    @pl.kernel(
        out_type=jax.ShapeDtypeStruct((B, S, D), x.dtype),
        mesh=vector_mesh,
        scratch_types=[
            pltpu.VMEM((_NBUF, B, _S_BLK, D), x.dtype),
            pltpu.VMEM((_NBUF, _S_BLK, D), x.dtype),
            pltpu.SemaphoreType.DMA((_NBUF,)),
            pltpu.SemaphoreType.DMA((_NBUF,)),
            pltpu.SemaphoreType.DMA((_NBUF,)),
        ],
    )
    def sc_add(x_hbm, emb_hbm, o_hbm, xbuf, ebuf, in_sems, emb_sems,
               out_sems):
        cid = lax.axis_index("core")
        sid = lax.axis_index("subcore")
        tid = cid * 16 + sid
        base_blk = tid * n_step

        def start_in(step, buf):
            row = (base_blk + step) * _S_BLK
            pltpu.make_async_copy(
                x_hbm.at[:, pl.ds(row, _S_BLK), :],
                xbuf.at[buf],
                in_sems.at[buf],
            ).start()
            pltpu.make_async_copy(
                emb_hbm.at[pl.ds(row, _S_BLK), :],
                ebuf.at[buf],
                emb_sems.at[buf],
            ).start()

        def wait_in(buf):
            pltpu.make_async_copy(
                x_hbm.at[:, pl.ds(0, _S_BLK), :], xbuf.at[buf],
                in_sems.at[buf],
            ).wait()
            pltpu.make_async_copy(
                emb_hbm.at[pl.ds(0, _S_BLK), :], ebuf.at[buf],
                emb_sems.at[buf],
            ).wait()

        def start_out(step, buf):
            row = (base_blk + step) * _S_BLK
            pltpu.make_async_copy(
                xbuf.at[buf],
                o_hbm.at[:, pl.ds(row, _S_BLK), :],
                out_sems.at[buf],
            ).start()

        def wait_out(buf):
            pltpu.make_async_copy(
                xbuf.at[buf], o_hbm.at[:, pl.ds(0, _S_BLK), :],
                out_sems.at[buf],
            ).wait()

        def compute(buf):
            @pl.loop(0, _S_BLK)
            def _(r):
                @plsc.parallel_loop(0, D, step=_LANES, unroll=8)
                def _(c):
                    e = ebuf.at[buf, r, pl.ds(c, _LANES)][...]
                    for b in range(B):
                        xbuf.at[buf, b, r, pl.ds(c, _LANES)][...] = (
                            xbuf.at[buf, b, r, pl.ds(c, _LANES)][...] + e
                        )

        start_in(0, 0)
        start_in(1, 1)
        wait_in(0)
        compute(0)
        start_out(0, 0)
        start_in(2, 2)

        @pl.loop(1, n_step)
        def _(s):
            buf = lax.rem(s, _NBUF)
            wait_in(buf)
            compute(buf)
            start_out(s, buf)
            nxt = s + 2
            nxt_buf = lax.rem(nxt, _NBUF)
            wait_out(nxt_buf)
            clamped = jnp.minimum(nxt, n_step - 1)
            start_in(clamped, nxt_buf)

        last_buf = lax.rem(n_step - 1, _NBUF)
        wait_out(last_buf)
        wait_in(lax.rem(n_step, _NBUF))
        wait_in(lax.rem(n_step + 1, _NBUF))

    return sc_add(x, pos)


# speedup vs baseline: 1.0717x; 1.0717x over previous
import jax
import jax.numpy as jnp
from jax.experimental import pallas as pl
from jax.experimental.pallas import tpu as pltpu
from jax.experimental.pallas import tpu_sc as plsc

_LANES = 16


def kernel(x, emb_table):
    B, S, D = x.shape
    pos = emb_table[:S]
    S_BLK = 4
    grid = (S // S_BLK,)
    buf_x = pl.Buffered(buffer_count=5)
    buf_e = pl.Buffered(buffer_count=3)

    vector_mesh = plsc.VectorSubcoreMesh(
        core_axis_name="core", subcore_axis_name="subcore"
    )

    @pl.kernel(out_type=jax.ShapeDtypeStruct((B, S, D), x.dtype),
               mesh=vector_mesh)
    def sc_add(x_hbm, emb_hbm, o_hbm):
        def body(x_vmem, emb_vmem, o_vmem):
            @pl.loop(0, S_BLK)
            def _(r):
                @plsc.parallel_loop(0, D, step=_LANES, unroll=8)
                def _(c):
                    e = emb_vmem.at[r, pl.ds(c, _LANES)][...]
                    for b in range(B):
                        o_vmem.at[b, r, pl.ds(c, _LANES)][...] = (
                            x_vmem.at[b, r, pl.ds(c, _LANES)][...] + e
                        )

        pltpu.emit_pipeline(
            body,
            grid=grid,
            in_specs=[
                pl.BlockSpec((B, S_BLK, D), lambda i: (0, i, 0),
                             pipeline_mode=buf_x),
                pl.BlockSpec((S_BLK, D), lambda i: (i, 0),
                             pipeline_mode=buf_e),
            ],
            out_specs=[pl.BlockSpec((B, S_BLK, D), lambda i: (0, i, 0))],
            core_axis_name=("core", "subcore"),
            dimension_semantics=(pltpu.PARALLEL,),
        )(x_hbm, emb_hbm, o_hbm)

    return sc_add(x, pos)
